# baseline (device time: 48121 ns/iter reference)
import jax
import jax.numpy as jnp
from jax import lax
from jax.experimental import pallas as pl
from jax.experimental.pallas import tpu as pltpu

N_DEV = 16
M_PER = 256
N_PER = 512
K = 4096

ACC_BOUND = 2011089.0
QSTEP = ACC_BOUND / 127.0


def kernel(x, w_mat, scale_x, scale_w):
    def body(x_ref, w_hbm, sx_ref, sw_ref, out_ref,
             w_vmem, y_send, comm_ref, w_sems, send_sems, recv_sems):
        me = lax.axis_index("i")
        scale = sx_ref[0] * sw_ref[0]

        x_val = x_ref[...]

        NPAIR = N_DEV // 2

        def start_w_copy(p):
            q = lax.rem(me // 2 + p, NPAIR)
            cp = pltpu.make_async_copy(
                w_hbm.at[:, pl.ds(q * 2 * N_PER, 2 * N_PER)],
                w_vmem.at[p % 2],
                w_sems.at[p % 2],
            )
            cp.start()
            return cp

        pending_w = start_w_copy(0)
        for p in range(NPAIR):
            nxt = start_w_copy(p + 1) if p + 1 < NPAIR else None
            pending_w.wait()
            acc = lax.dot_general(
                x_val, w_vmem[p % 2],
                (((1,), (0,)), ((), ())),
                preferred_element_type=jnp.int32,
            )
            yq = jnp.clip(
                jnp.round(acc.astype(jnp.float32) * (1.0 / QSTEP)),
                -127.0, 127.0,
            ).astype(jnp.int8)
            q = lax.rem(me // 2 + p, NPAIR)
            for h in range(2):
                slot = 2 * p + h
                t = 2 * q + h
                y_send[slot] = yq[:, h * N_PER:(h + 1) * N_PER]

                @pl.when(t == me)
                def _():
                    comm_ref[pl.ds(me * M_PER, M_PER), :] = y_send[slot]

                @pl.when(t != me)
                def _():
                    pltpu.make_async_remote_copy(
                        src_ref=y_send.at[slot],
                        dst_ref=comm_ref.at[pl.ds(me * M_PER, M_PER), :],
                        send_sem=send_sems.at[slot],
                        recv_sem=recv_sems.at[me],
                        device_id=t,
                        device_id_type=pl.DeviceIdType.LOGICAL,
                    ).start()
            pending_w = nxt

        for i in range(N_DEV):
            @pl.when(i != me)
            def _():
                dummy = pltpu.make_async_remote_copy(
                    src_ref=y_send.at[0],
                    dst_ref=comm_ref.at[pl.ds(i * M_PER, M_PER), :],
                    send_sem=send_sems.at[0],
                    recv_sem=recv_sems.at[i],
                    device_id=me,
                    device_id_type=pl.DeviceIdType.LOGICAL,
                )
                dummy.wait_recv()
        own_slot = lax.rem(me, 2)
        for s in range(N_DEV):
            @pl.when(s != own_slot)
            def _():
                pltpu.make_async_remote_copy(
                    src_ref=y_send.at[s],
                    dst_ref=comm_ref.at[pl.ds(me * M_PER, M_PER), :],
                    send_sem=send_sems.at[s],
                    recv_sem=recv_sems.at[me],
                    device_id=me,
                    device_id_type=pl.DeviceIdType.LOGICAL,
                ).wait_send()

        out_ref[...] = comm_ref[...].astype(jnp.float32) * (QSTEP * scale)

    return pl.pallas_call(
        body,
        out_shape=jax.ShapeDtypeStruct((N_DEV * M_PER, N_PER), jnp.float32),
        in_specs=[
            pl.BlockSpec(memory_space=pltpu.VMEM),
            pl.BlockSpec(memory_space=pltpu.MemorySpace.HBM),
            pl.BlockSpec(memory_space=pltpu.SMEM),
            pl.BlockSpec(memory_space=pltpu.SMEM),
        ],
        out_specs=pl.BlockSpec(memory_space=pltpu.VMEM),
        scratch_shapes=[
            pltpu.VMEM((2, K, 2 * N_PER), jnp.int8),
            pltpu.VMEM((N_DEV, M_PER, N_PER), jnp.int8),
            pltpu.VMEM((N_DEV * M_PER, N_PER), jnp.int8),
            pltpu.SemaphoreType.DMA((2,)),
            pltpu.SemaphoreType.DMA((N_DEV,)),
            pltpu.SemaphoreType.DMA((N_DEV,)),
        ],
    )(x, w_mat, scale_x, scale_w)


# device time: 46250 ns/iter; 1.0405x vs baseline; 1.0405x over previous
import jax
import jax.numpy as jnp
from jax import lax
from jax.experimental import pallas as pl
from jax.experimental.pallas import tpu as pltpu

N_DEV = 16
M_PER = 256
N_PER = 512
K = 4096

ACC_BOUND = 2011089.0
QSTEP = ACC_BOUND / 127.0


def kernel(x, w_mat, scale_x, scale_w):
    def body(x_ref, w_hbm, sx_ref, sw_ref, out_ref,
             w_vmem, y_send, comm_ref, w_sems, send_sems, recv_sems):
        me = lax.axis_index("i")
        scale = sx_ref[0] * sw_ref[0]

        def start_w_copy(s):
            t = lax.rem(me + s, N_DEV)
            cp = pltpu.make_async_copy(
                w_hbm.at[:, pl.ds(t * N_PER, N_PER)],
                w_vmem.at[s % 2],
                w_sems.at[s % 2],
            )
            cp.start()
            return cp

        pending_w = start_w_copy(0)
        for s in range(N_DEV):
            nxt = start_w_copy(s + 1) if s + 1 < N_DEV else None
            pending_w.wait()
            acc = lax.dot_general(
                x_ref[...], w_vmem[s % 2],
                (((1,), (0,)), ((), ())),
                preferred_element_type=jnp.int32,
            )
            y = jnp.clip(
                jnp.round(acc.astype(jnp.float32) * (1.0 / QSTEP)),
                -127.0, 127.0,
            ).astype(jnp.int8)
            if s == 0:
                out_ref[pl.ds(me * M_PER, M_PER), :] = (
                    acc.astype(jnp.float32) * scale)
            else:
                t = lax.rem(me + s, N_DEV)
                y_send[s] = y
                rdma = pltpu.make_async_remote_copy(
                    src_ref=y_send.at[s],
                    dst_ref=comm_ref.at[pl.ds(me * M_PER, M_PER), :],
                    send_sem=send_sems.at[s],
                    recv_sem=recv_sems.at[me],
                    device_id=t,
                    device_id_type=pl.DeviceIdType.LOGICAL,
                )
                rdma.start()
            pending_w = nxt

        dq = QSTEP * scale
        for k in range(1, N_DEV):
            i = lax.rem(me - k + N_DEV, N_DEV)
            pltpu.make_async_remote_copy(
                src_ref=y_send.at[0],
                dst_ref=comm_ref.at[pl.ds(i * M_PER, M_PER), :],
                send_sem=send_sems.at[0],
                recv_sem=recv_sems.at[i],
                device_id=me,
                device_id_type=pl.DeviceIdType.LOGICAL,
            ).wait_recv()
            out_ref[pl.ds(i * M_PER, M_PER), :] = (
                comm_ref[pl.ds(i * M_PER, M_PER), :].astype(jnp.float32) * dq)

        for s in range(1, N_DEV):
            pltpu.make_async_remote_copy(
                src_ref=y_send.at[s],
                dst_ref=comm_ref.at[pl.ds(me * M_PER, M_PER), :],
                send_sem=send_sems.at[s],
                recv_sem=recv_sems.at[me],
                device_id=me,
                device_id_type=pl.DeviceIdType.LOGICAL,
            ).wait_send()

    return pl.pallas_call(
        body,
        out_shape=jax.ShapeDtypeStruct((N_DEV * M_PER, N_PER), jnp.float32),
        in_specs=[
            pl.BlockSpec(memory_space=pltpu.VMEM),
            pl.BlockSpec(memory_space=pltpu.MemorySpace.HBM),
            pl.BlockSpec(memory_space=pltpu.SMEM),
            pl.BlockSpec(memory_space=pltpu.SMEM),
        ],
        out_specs=pl.BlockSpec(memory_space=pltpu.VMEM),
        scratch_shapes=[
            pltpu.VMEM((2, K, N_PER), jnp.int8),
            pltpu.VMEM((N_DEV, M_PER, N_PER), jnp.int8),
            pltpu.VMEM((N_DEV * M_PER, N_PER), jnp.int8),
            pltpu.SemaphoreType.DMA((2,)),
            pltpu.SemaphoreType.DMA((N_DEV,)),
            pltpu.SemaphoreType.DMA((N_DEV,)),
        ],
    )(x, w_mat, scale_x, scale_w)
